# baseline (speedup 1.0000x reference)
import functools

import numpy as np
import jax
import jax.numpy as jnp
from jax import lax
from jax.experimental import pallas as pl
from jax.experimental.pallas import tpu as pltpu
from jax.experimental.pallas import tpu_sc as plsc

_DIM = 512
_VOCAB = 28
_MAX_CHARGE = 10
_B, _N, _L = 32, 16, 64
_ROWS = _B * _N * _L
_TROW = 32
_TSIZE = _TROW * (_B + 1)


def _terms(n):
    base = 0.001 / (2.0 * np.pi)
    scale = 10000.0 / 0.001
    return base * scale ** (np.arange(n, dtype=np.float64) / (n - 1))


_R64 = (1.0 / _terms(64).astype(np.float32)).astype(np.float32).reshape(1, 64)
_R128 = (1.0 / _terms(128).astype(np.float32)).astype(np.float32).reshape(1, 128)


def _table_kernel(aa32_ref, idx2m32_ref, charge_ref, mass_col_ref,
                  charge_col_ref, r64_ref, r128_ref, tab_ref):
    targ = idx2m32_ref[...] * r64_ref[...]
    s = jnp.sin(targ)
    c = jnp.cos(targ)
    aa32 = aa32_ref[...]
    tab_ref[0:_TROW, 0:256] = aa32
    tab_ref[0:_TROW, 256:320] = s
    tab_ref[0:_TROW, 320:384] = c
    tab_ref[0:_TROW, 384:512] = jnp.zeros((_TROW, 128), jnp.float32)

    marg = mass_col_ref[...] * r128_ref[...]
    cidx = charge_col_ref[...].astype(jnp.int32) - 1
    coh = (lax.broadcasted_iota(jnp.int32, (_B, _MAX_CHARGE), 1)
           == cidx).astype(jnp.float32)
    crow = lax.dot_general(coh, charge_ref[...], (((1,), (0,)), ((), ())),
                           precision=lax.Precision.HIGHEST,
                           preferred_element_type=jnp.float32)
    prec256 = jnp.concatenate([jnp.sin(marg), jnp.cos(marg)], axis=1) + crow
    cp1 = c + 1.0
    for b in range(_B):
        r0 = _TROW * (b + 1)
        tab_ref[pl.ds(r0, _TROW), 0:256] = aa32 + prec256[b:b + 1, :]
        tab_ref[pl.ds(r0, _TROW), 256:320] = s
        tab_ref[pl.ds(r0, _TROW), 320:384] = cp1
        tab_ref[pl.ds(r0, _TROW), 384:512] = jnp.zeros((_TROW, 128),
                                                       jnp.float32)


def _build_table(aa32, idx2m32, charge_table, mass_col, charge_col):
    return pl.pallas_call(
        _table_kernel,
        grid=(1,),
        in_specs=[
            pl.BlockSpec((_TROW, 256), lambda i: (0, 0)),
            pl.BlockSpec((_TROW, 1), lambda i: (0, 0)),
            pl.BlockSpec((_MAX_CHARGE, 256), lambda i: (0, 0)),
            pl.BlockSpec((_B, 1), lambda i: (0, 0)),
            pl.BlockSpec((_B, 1), lambda i: (0, 0)),
            pl.BlockSpec((1, 64), lambda i: (0, 0)),
            pl.BlockSpec((1, 128), lambda i: (0, 0)),
        ],
        out_specs=pl.BlockSpec((_TSIZE, _DIM), lambda i: (0, 0)),
        out_shape=jax.ShapeDtypeStruct((_TSIZE, _DIM), jnp.float32),
    )(aa32, idx2m32, charge_table, mass_col, charge_col,
      jnp.asarray(_R64), jnp.asarray(_R128))


_INFO = plsc.get_sparse_core_info()
_NW = _INFO.num_cores * _INFO.num_subcores
_RPW = _ROWS // _NW
_CHUNK = 64
_NCH = _RPW // _CHUNK


@functools.partial(
    pl.kernel,
    mesh=plsc.VectorSubcoreMesh(core_axis_name="c", subcore_axis_name="s"),
    out_type=jax.ShapeDtypeStruct((_ROWS, _DIM), jnp.float32),
    scratch_types=[
        pltpu.VMEM((2, _CHUNK), jnp.int32),
        pltpu.VMEM((_CHUNK, _DIM), jnp.float32),
        pltpu.VMEM((_CHUNK, _DIM), jnp.float32),
        pltpu.SemaphoreType.DMA,
        pltpu.SemaphoreType.DMA,
        pltpu.SemaphoreType.DMA,
        pltpu.SemaphoreType.DMA,
    ],
)
def _sc_gather(tab_hbm, tok_hbm, out_hbm, idx_v, rows0, rows1, g0, g1, w0, w1):
    wid = lax.axis_index("s") * _INFO.num_cores + lax.axis_index("c")
    base = wid * _RPW
    kadd = _TROW * (wid + 1)
    lane = lax.broadcasted_iota(jnp.int32, (16,), 0)
    rows = (rows0, rows1)
    gsem = (g0, g1)
    wsem = (w0, w1)
    g_h = [None, None]
    w_h = [None, None]

    for c in range(_NCH):
        b = c & 1
        if c >= 2:
            w_h[b].wait()
        r0 = base + c * _CHUNK
        pltpu.sync_copy(tok_hbm.at[pl.ds(r0, _CHUNK)], idx_v.at[b])
        v = idx_v[b, pl.ds(0, 16)]
        idx_v[b, pl.ds(0, 16)] = jnp.where(lane == 0, v + kadd, v)
        g_h[b] = pltpu.async_copy(tab_hbm.at[idx_v.at[b]], rows[b], gsem[b])
        if c >= 1:
            bb = (c - 1) & 1
            g_h[bb].wait()
            rp = base + (c - 1) * _CHUNK
            w_h[bb] = pltpu.async_copy(rows[bb],
                                       out_hbm.at[pl.ds(rp, _CHUNK)], wsem[bb])
    bl = (_NCH - 1) & 1
    g_h[bl].wait()
    w_h[bl] = pltpu.async_copy(rows[bl],
                               out_hbm.at[pl.ds(base + (_NCH - 1) * _CHUNK,
                                                _CHUNK)], wsem[bl])
    w_h[0].wait()
    w_h[1].wait()


def _suffix_kernel(out_in_ref, mass_cols_ref, tok_lf_ref, idx2m_s_ref,
                   r64_ref, out_ref, suf_scr, scan_scr):
    b = pl.program_id(0)

    @pl.when(b == 0)
    def _prologue():
        tok_lf = tok_lf_ref[...]
        pm_all = jnp.zeros((64, 512), jnp.float32)
        for v in range(_VOCAB):
            pm_all = jnp.where(tok_lf == v, idx2m_s_ref[v], pm_all)
        scan_scr[...] = pm_all

        def _scan_body(l, carry):
            scan_scr[pl.ds(l, 1), :] = (scan_scr[pl.ds(l, 1), :]
                                        + scan_scr[pl.ds(l - 1, 1), :])
            return carry

        lax.fori_loop(1, 64, _scan_body, 0, unroll=True)
        suf_all = mass_cols_ref[...] - scan_scr[...]
        suf_eo = suf_all.reshape(32, 2, 512).transpose(1, 0, 2).reshape(64, 512)
        suf_scr[...] = suf_eo.T

    sfp = suf_scr[pl.ds(b * 16, 16), :]
    r64 = r64_ref[...][None, :, :]
    arg_p = jnp.concatenate([sfp[:, 0:32, None] * r64,
                             sfp[:, 32:64, None] * r64], axis=2)
    sin_p, cos_p = jnp.sin(arg_p), jnp.cos(arg_p)
    enc_e = jnp.concatenate([sin_p[:, :, 0:64], cos_p[:, :, 0:64]], axis=2)
    enc_o = jnp.concatenate([sin_p[:, :, 64:128], cos_p[:, :, 64:128]], axis=2)
    lpos = lax.broadcasted_iota(jnp.int32, (16, 32, 128), 1)
    lane = lax.broadcasted_iota(jnp.int32, (16, 32, 128), 2)
    enc_e = enc_e + jnp.where((lpos == 0) & (lane >= 64), 1.0, 0.0)
    enc = jnp.stack([enc_e, enc_o], axis=2).reshape(16, 64, 128)
    out_ref[...] = enc.reshape(1024, 128)


def _suffix_pass(out_flat, mass_cols, tok_lf, idx_to_mass):
    return pl.pallas_call(
        _suffix_kernel,
        grid=(_B,),
        in_specs=[
            pl.BlockSpec(memory_space=pl.ANY),
            pl.BlockSpec((1, _B * _N), lambda b: (0, 0)),
            pl.BlockSpec((_L, _B * _N), lambda b: (0, 0)),
            pl.BlockSpec(memory_space=pltpu.SMEM),
            pl.BlockSpec((1, 64), lambda b: (0, 0)),
        ],
        out_specs=pl.BlockSpec((_N * _L, 128), lambda b: (b, 3)),
        out_shape=jax.ShapeDtypeStruct((_ROWS, _DIM), jnp.float32),
        scratch_shapes=[pltpu.VMEM((_B * _N, _L), jnp.float32),
                        pltpu.VMEM((_L, _B * _N), jnp.float32)],
        input_output_aliases={0: 0},
    )(out_flat, mass_cols, tok_lf, idx_to_mass, jnp.asarray(_R64))


def kernel(tokens, precursors, aa_table, charge_table, idx_to_mass):
    B, N, L = tokens.shape
    tok_lf = tokens.transpose(2, 0, 1).reshape(L, B * N)
    mass_cols = jnp.repeat(precursors[:, 0], N).reshape(1, B * N)
    mass_col = precursors[:, 0].reshape(B, 1)
    charge_col = precursors[:, 1].reshape(B, 1)
    aa32 = jnp.pad(aa_table, ((0, _TROW - _VOCAB), (0, 0)))
    idx2m32 = jnp.pad(idx_to_mass, (0, _TROW - _VOCAB)).reshape(_TROW, 1)

    tab = _build_table(aa32, idx2m32, charge_table, mass_col, charge_col)
    out_flat = _sc_gather(tab, tokens.reshape(_ROWS))
    out_flat = _suffix_pass(out_flat, mass_cols, tok_lf, idx_to_mass)
    return out_flat.reshape(B, N, L, _DIM)


# speedup vs baseline: 1.1200x; 1.1200x over previous
import functools

import numpy as np
import jax
import jax.numpy as jnp
from jax import lax
from jax.experimental import pallas as pl
from jax.experimental.pallas import tpu as pltpu
from jax.experimental.pallas import tpu_sc as plsc

_DIM = 512
_VOCAB = 28
_MAX_CHARGE = 10
_B, _N, _L = 32, 16, 64
_ROWS = _B * _N * _L
_TROW = 32
_TSIZE = _TROW * (_B + 1)


def _terms(n):
    base = 0.001 / (2.0 * np.pi)
    scale = 10000.0 / 0.001
    return base * scale ** (np.arange(n, dtype=np.float64) / (n - 1))


_R64 = (1.0 / _terms(64).astype(np.float32)).astype(np.float32).reshape(1, 64)
_R128 = (1.0 / _terms(128).astype(np.float32)).astype(np.float32).reshape(1, 128)


_GW = 384


def _table_kernel(aa32_ref, idx2m32_ref, charge_ref, mass_col_ref,
                  charge_col_ref, r64_ref, r128_ref, tab_ref):
    targ = idx2m32_ref[...] * r64_ref[...]
    s = jnp.sin(targ)
    c = jnp.cos(targ)
    aa32 = aa32_ref[...]
    tab_ref[0:_TROW, 0:256] = aa32
    tab_ref[0:_TROW, 256:320] = s
    tab_ref[0:_TROW, 320:384] = c

    marg = mass_col_ref[...] * r128_ref[...]
    cidx = charge_col_ref[...].astype(jnp.int32) - 1
    coh = (lax.broadcasted_iota(jnp.int32, (_B, _MAX_CHARGE), 1)
           == cidx).astype(jnp.float32)
    crow = lax.dot_general(coh, charge_ref[...], (((1,), (0,)), ((), ())),
                           precision=lax.Precision.HIGHEST,
                           preferred_element_type=jnp.float32)
    prec256 = jnp.concatenate([jnp.sin(marg), jnp.cos(marg)], axis=1) + crow
    cp1 = c + 1.0
    for b in range(_B):
        r0 = _TROW * (b + 1)
        tab_ref[pl.ds(r0, _TROW), 0:256] = aa32 + prec256[b:b + 1, :]
        tab_ref[pl.ds(r0, _TROW), 256:320] = s
        tab_ref[pl.ds(r0, _TROW), 320:384] = cp1


def _build_table(aa32, idx2m32, charge_table, mass_col, charge_col):
    return pl.pallas_call(
        _table_kernel,
        grid=(1,),
        in_specs=[
            pl.BlockSpec((_TROW, 256), lambda i: (0, 0)),
            pl.BlockSpec((_TROW, 1), lambda i: (0, 0)),
            pl.BlockSpec((_MAX_CHARGE, 256), lambda i: (0, 0)),
            pl.BlockSpec((_B, 1), lambda i: (0, 0)),
            pl.BlockSpec((_B, 1), lambda i: (0, 0)),
            pl.BlockSpec((1, 64), lambda i: (0, 0)),
            pl.BlockSpec((1, 128), lambda i: (0, 0)),
        ],
        out_specs=pl.BlockSpec((_TSIZE, _GW), lambda i: (0, 0)),
        out_shape=jax.ShapeDtypeStruct((_TSIZE, _GW), jnp.float32),
    )(aa32, idx2m32, charge_table, mass_col, charge_col,
      jnp.asarray(_R64), jnp.asarray(_R128))


_INFO = plsc.get_sparse_core_info()
_NW = _INFO.num_cores * _INFO.num_subcores
_RPW = _ROWS // _NW
_CHUNK = 64
_NCH = _RPW // _CHUNK


@functools.partial(
    pl.kernel,
    mesh=plsc.VectorSubcoreMesh(core_axis_name="c", subcore_axis_name="s"),
    out_type=jax.ShapeDtypeStruct((_ROWS, _DIM), jnp.float32),
    scratch_types=[
        pltpu.VMEM((2, _CHUNK), jnp.int32),
        pltpu.VMEM((_CHUNK, _GW), jnp.float32),
        pltpu.VMEM((_CHUNK, _GW), jnp.float32),
        pltpu.SemaphoreType.DMA,
        pltpu.SemaphoreType.DMA,
        pltpu.SemaphoreType.DMA,
        pltpu.SemaphoreType.DMA,
    ],
)
def _sc_gather(tab_hbm, tok_hbm, out_hbm, idx_v, rows0, rows1, g0, g1, w0, w1):
    wid = lax.axis_index("s") * _INFO.num_cores + lax.axis_index("c")
    base = wid * _RPW
    kadd = _TROW * (wid + 1)
    lane = lax.broadcasted_iota(jnp.int32, (16,), 0)
    rows = (rows0, rows1)
    gsem = (g0, g1)
    wsem = (w0, w1)
    g_h = [None, None]
    w_h = [None, None]

    for c in range(_NCH):
        b = c & 1
        if c >= 2:
            w_h[b].wait()
        r0 = base + c * _CHUNK
        pltpu.sync_copy(tok_hbm.at[pl.ds(r0, _CHUNK)], idx_v.at[b])
        v = idx_v[b, pl.ds(0, 16)]
        idx_v[b, pl.ds(0, 16)] = jnp.where(lane == 0, v + kadd, v)
        g_h[b] = pltpu.async_copy(tab_hbm.at[idx_v.at[b]], rows[b], gsem[b])
        if c >= 1:
            bb = (c - 1) & 1
            g_h[bb].wait()
            rp = base + (c - 1) * _CHUNK
            w_h[bb] = pltpu.async_copy(
                rows[bb], out_hbm.at[pl.ds(rp, _CHUNK), pl.ds(0, _GW)],
                wsem[bb])
    bl = (_NCH - 1) & 1
    g_h[bl].wait()
    w_h[bl] = pltpu.async_copy(
        rows[bl], out_hbm.at[pl.ds(base + (_NCH - 1) * _CHUNK, _CHUNK),
                             pl.ds(0, _GW)], wsem[bl])
    w_h[0].wait()
    w_h[1].wait()


def _suffix_kernel(out_in_ref, mass_cols_ref, tok_lf_ref, idx2m_s_ref,
                   r64_ref, out_ref, suf_scr, scan_scr):
    b = pl.program_id(0)

    @pl.when(b == 0)
    def _prologue():
        tok_lf = tok_lf_ref[...]
        pm_all = jnp.zeros((64, 512), jnp.float32)
        for v in range(_VOCAB):
            pm_all = jnp.where(tok_lf == v, idx2m_s_ref[v], pm_all)
        scan_scr[...] = pm_all

        def _scan_body(l, carry):
            scan_scr[pl.ds(l, 1), :] = (scan_scr[pl.ds(l, 1), :]
                                        + scan_scr[pl.ds(l - 1, 1), :])
            return carry

        lax.fori_loop(1, 64, _scan_body, 0, unroll=True)
        suf_all = mass_cols_ref[...] - scan_scr[...]
        suf_eo = suf_all.reshape(32, 2, 512).transpose(1, 0, 2).reshape(64, 512)
        suf_scr[...] = suf_eo.T

    sfp = suf_scr[pl.ds(b * 16, 16), :]
    r64 = r64_ref[...][None, :, :]
    arg_p = jnp.concatenate([sfp[:, 0:32, None] * r64,
                             sfp[:, 32:64, None] * r64], axis=2)
    sin_p, cos_p = jnp.sin(arg_p), jnp.cos(arg_p)
    enc_e = jnp.concatenate([sin_p[:, :, 0:64], cos_p[:, :, 0:64]], axis=2)
    enc_o = jnp.concatenate([sin_p[:, :, 64:128], cos_p[:, :, 64:128]], axis=2)
    lpos = lax.broadcasted_iota(jnp.int32, (16, 32, 128), 1)
    lane = lax.broadcasted_iota(jnp.int32, (16, 32, 128), 2)
    enc_e = enc_e + jnp.where((lpos == 0) & (lane >= 64), 1.0, 0.0)
    enc = jnp.stack([enc_e, enc_o], axis=2).reshape(16, 64, 128)
    out_ref[...] = enc.reshape(1024, 128)


def _suffix_pass(out_flat, mass_cols, tok_lf, idx_to_mass):
    return pl.pallas_call(
        _suffix_kernel,
        grid=(_B,),
        in_specs=[
            pl.BlockSpec(memory_space=pl.ANY),
            pl.BlockSpec((1, _B * _N), lambda b: (0, 0)),
            pl.BlockSpec((_L, _B * _N), lambda b: (0, 0)),
            pl.BlockSpec(memory_space=pltpu.SMEM),
            pl.BlockSpec((1, 64), lambda b: (0, 0)),
        ],
        out_specs=pl.BlockSpec((_N * _L, 128), lambda b: (b, 3)),
        out_shape=jax.ShapeDtypeStruct((_ROWS, _DIM), jnp.float32),
        scratch_shapes=[pltpu.VMEM((_B * _N, _L), jnp.float32),
                        pltpu.VMEM((_L, _B * _N), jnp.float32)],
        input_output_aliases={0: 0},
    )(out_flat, mass_cols, tok_lf, idx_to_mass, jnp.asarray(_R64))


def kernel(tokens, precursors, aa_table, charge_table, idx_to_mass):
    B, N, L = tokens.shape
    tok_lf = tokens.transpose(2, 0, 1).reshape(L, B * N)
    mass_cols = jnp.repeat(precursors[:, 0], N).reshape(1, B * N)
    mass_col = precursors[:, 0].reshape(B, 1)
    charge_col = precursors[:, 1].reshape(B, 1)
    aa32 = jnp.pad(aa_table, ((0, _TROW - _VOCAB), (0, 0)))
    idx2m32 = jnp.pad(idx_to_mass, (0, _TROW - _VOCAB)).reshape(_TROW, 1)

    tab = _build_table(aa32, idx2m32, charge_table, mass_col, charge_col)
    out_flat = _sc_gather(tab, tokens.reshape(_ROWS))
    out_flat = _suffix_pass(out_flat, mass_cols, tok_lf, idx_to_mass)
    return out_flat.reshape(B, N, L, _DIM)


# speedup vs baseline: 1.1272x; 1.0064x over previous
import functools

import numpy as np
import jax
import jax.numpy as jnp
from jax import lax
from jax.experimental import pallas as pl
from jax.experimental.pallas import tpu as pltpu
from jax.experimental.pallas import tpu_sc as plsc

_DIM = 512
_VOCAB = 28
_MAX_CHARGE = 10
_B, _N, _L = 32, 16, 64
_ROWS = _B * _N * _L
_TROW = 32
_TSIZE = _TROW * (_B + 1)


def _terms(n):
    base = 0.001 / (2.0 * np.pi)
    scale = 10000.0 / 0.001
    return base * scale ** (np.arange(n, dtype=np.float64) / (n - 1))


_R64 = (1.0 / _terms(64).astype(np.float32)).astype(np.float32).reshape(1, 64)
_R128 = (1.0 / _terms(128).astype(np.float32)).astype(np.float32).reshape(1, 128)


_GW = 384


def _table_kernel(aa32_ref, idx2m32_ref, charge_ref, mass_col_ref,
                  charge_col_ref, r64_ref, r128_ref, tab_ref):
    targ = idx2m32_ref[...] * r64_ref[...]
    s = jnp.sin(targ)
    c = jnp.cos(targ)
    aa32 = aa32_ref[...]
    tab_ref[0:_TROW, 0:256] = aa32
    tab_ref[0:_TROW, 256:320] = s
    tab_ref[0:_TROW, 320:384] = c

    marg = mass_col_ref[...] * r128_ref[...]
    cidx = charge_col_ref[...].astype(jnp.int32) - 1
    coh = (lax.broadcasted_iota(jnp.int32, (_B, _MAX_CHARGE), 1)
           == cidx).astype(jnp.float32)
    crow = lax.dot_general(coh, charge_ref[...], (((1,), (0,)), ((), ())),
                           precision=lax.Precision.HIGHEST,
                           preferred_element_type=jnp.float32)
    prec256 = jnp.concatenate([jnp.sin(marg), jnp.cos(marg)], axis=1) + crow
    cp1 = c + 1.0
    for b in range(_B):
        r0 = _TROW * (b + 1)
        tab_ref[pl.ds(r0, _TROW), 0:256] = aa32 + prec256[b:b + 1, :]
        tab_ref[pl.ds(r0, _TROW), 256:320] = s
        tab_ref[pl.ds(r0, _TROW), 320:384] = cp1


def _build_table(aa32, idx2m32, charge_table, mass_col, charge_col):
    return pl.pallas_call(
        _table_kernel,
        grid=(1,),
        in_specs=[
            pl.BlockSpec((_TROW, 256), lambda i: (0, 0)),
            pl.BlockSpec((_TROW, 1), lambda i: (0, 0)),
            pl.BlockSpec((_MAX_CHARGE, 256), lambda i: (0, 0)),
            pl.BlockSpec((_B, 1), lambda i: (0, 0)),
            pl.BlockSpec((_B, 1), lambda i: (0, 0)),
            pl.BlockSpec((1, 64), lambda i: (0, 0)),
            pl.BlockSpec((1, 128), lambda i: (0, 0)),
        ],
        out_specs=pl.BlockSpec((_TSIZE, _GW), lambda i: (0, 0)),
        out_shape=jax.ShapeDtypeStruct((_TSIZE, _GW), jnp.float32),
    )(aa32, idx2m32, charge_table, mass_col, charge_col,
      jnp.asarray(_R64), jnp.asarray(_R128))


_INFO = plsc.get_sparse_core_info()
_NW = _INFO.num_cores * _INFO.num_subcores
_RPW = _ROWS // _NW
_CHUNK = 128
_NCH = _RPW // _CHUNK


@functools.partial(
    pl.kernel,
    mesh=plsc.VectorSubcoreMesh(core_axis_name="c", subcore_axis_name="s"),
    out_type=jax.ShapeDtypeStruct((_ROWS, _DIM), jnp.float32),
    scratch_types=[
        pltpu.VMEM((2, _CHUNK), jnp.int32),
        pltpu.VMEM((_CHUNK, _GW), jnp.float32),
        pltpu.VMEM((_CHUNK, _GW), jnp.float32),
        pltpu.SemaphoreType.DMA,
        pltpu.SemaphoreType.DMA,
        pltpu.SemaphoreType.DMA,
        pltpu.SemaphoreType.DMA,
    ],
)
def _sc_gather(tab_hbm, tok_hbm, out_hbm, idx_v, rows0, rows1, g0, g1, w0, w1):
    wid = lax.axis_index("s") * _INFO.num_cores + lax.axis_index("c")
    base = wid * _RPW
    kadd = _TROW * (wid + 1)
    lane = lax.broadcasted_iota(jnp.int32, (16,), 0)
    rows = (rows0, rows1)
    gsem = (g0, g1)
    wsem = (w0, w1)
    g_h = [None, None]
    w_h = [None, None]

    for c in range(_NCH):
        b = c & 1
        if c >= 2:
            w_h[b].wait()
        r0 = base + c * _CHUNK
        pltpu.sync_copy(tok_hbm.at[pl.ds(r0, _CHUNK)], idx_v.at[b])
        for p in range(0, _CHUNK, _L):
            v = idx_v[b, pl.ds(p, 16)]
            idx_v[b, pl.ds(p, 16)] = jnp.where(lane == 0, v + kadd, v)
        g_h[b] = pltpu.async_copy(tab_hbm.at[idx_v.at[b]], rows[b], gsem[b])
        if c >= 1:
            bb = (c - 1) & 1
            g_h[bb].wait()
            rp = base + (c - 1) * _CHUNK
            w_h[bb] = pltpu.async_copy(
                rows[bb], out_hbm.at[pl.ds(rp, _CHUNK), pl.ds(0, _GW)],
                wsem[bb])
    bl = (_NCH - 1) & 1
    g_h[bl].wait()
    w_h[bl] = pltpu.async_copy(
        rows[bl], out_hbm.at[pl.ds(base + (_NCH - 1) * _CHUNK, _CHUNK),
                             pl.ds(0, _GW)], wsem[bl])
    w_h[0].wait()
    w_h[1].wait()


def _suffix_kernel(out_in_ref, mass_cols_ref, tok_lf_ref, idx2m_s_ref,
                   r64_ref, out_ref, suf_scr, scan_scr):
    b = pl.program_id(0)

    @pl.when(b == 0)
    def _prologue():
        tok_lf = tok_lf_ref[...]
        pm_all = jnp.zeros((64, 512), jnp.float32)
        for v in range(_VOCAB):
            pm_all = jnp.where(tok_lf == v, idx2m_s_ref[v], pm_all)
        scan_scr[...] = pm_all

        def _scan_body(l, carry):
            scan_scr[pl.ds(l, 1), :] = (scan_scr[pl.ds(l, 1), :]
                                        + scan_scr[pl.ds(l - 1, 1), :])
            return carry

        lax.fori_loop(1, 64, _scan_body, 0, unroll=True)
        suf_all = mass_cols_ref[...] - scan_scr[...]
        suf_eo = suf_all.reshape(32, 2, 512).transpose(1, 0, 2).reshape(64, 512)
        suf_scr[...] = suf_eo.T

    sfp = suf_scr[pl.ds(b * 16, 16), :]
    r64 = r64_ref[...][None, :, :]
    arg_p = jnp.concatenate([sfp[:, 0:32, None] * r64,
                             sfp[:, 32:64, None] * r64], axis=2)
    sin_p, cos_p = jnp.sin(arg_p), jnp.cos(arg_p)
    enc_e = jnp.concatenate([sin_p[:, :, 0:64], cos_p[:, :, 0:64]], axis=2)
    enc_o = jnp.concatenate([sin_p[:, :, 64:128], cos_p[:, :, 64:128]], axis=2)
    lpos = lax.broadcasted_iota(jnp.int32, (16, 32, 128), 1)
    lane = lax.broadcasted_iota(jnp.int32, (16, 32, 128), 2)
    enc_e = enc_e + jnp.where((lpos == 0) & (lane >= 64), 1.0, 0.0)
    enc = jnp.stack([enc_e, enc_o], axis=2).reshape(16, 64, 128)
    out_ref[...] = enc.reshape(1024, 128)


def _suffix_pass(out_flat, mass_cols, tok_lf, idx_to_mass):
    return pl.pallas_call(
        _suffix_kernel,
        grid=(_B,),
        in_specs=[
            pl.BlockSpec(memory_space=pl.ANY),
            pl.BlockSpec((1, _B * _N), lambda b: (0, 0)),
            pl.BlockSpec((_L, _B * _N), lambda b: (0, 0)),
            pl.BlockSpec(memory_space=pltpu.SMEM),
            pl.BlockSpec((1, 64), lambda b: (0, 0)),
        ],
        out_specs=pl.BlockSpec((_N * _L, 128), lambda b: (b, 3)),
        out_shape=jax.ShapeDtypeStruct((_ROWS, _DIM), jnp.float32),
        scratch_shapes=[pltpu.VMEM((_B * _N, _L), jnp.float32),
                        pltpu.VMEM((_L, _B * _N), jnp.float32)],
        input_output_aliases={0: 0},
    )(out_flat, mass_cols, tok_lf, idx_to_mass, jnp.asarray(_R64))


def kernel(tokens, precursors, aa_table, charge_table, idx_to_mass):
    B, N, L = tokens.shape
    tok_lf = tokens.transpose(2, 0, 1).reshape(L, B * N)
    mass_cols = jnp.repeat(precursors[:, 0], N).reshape(1, B * N)
    mass_col = precursors[:, 0].reshape(B, 1)
    charge_col = precursors[:, 1].reshape(B, 1)
    aa32 = jnp.pad(aa_table, ((0, _TROW - _VOCAB), (0, 0)))
    idx2m32 = jnp.pad(idx_to_mass, (0, _TROW - _VOCAB)).reshape(_TROW, 1)

    tab = _build_table(aa32, idx2m32, charge_table, mass_col, charge_col)
    out_flat = _sc_gather(tab, tokens.reshape(_ROWS))
    out_flat = _suffix_pass(out_flat, mass_cols, tok_lf, idx_to_mass)
    return out_flat.reshape(B, N, L, _DIM)
